# baseline (speedup 1.0000x reference)
import functools

import jax
import jax.numpy as jnp
from jax import lax
from jax.experimental import pallas as pl
from jax.experimental.pallas import tpu as pltpu
from jax.experimental.pallas import tpu_sc as plsc

VOCAB = 100000
CTX = 512
D = 8

_NC, _NS = 2, 16
_NW = _NC * _NS
_TOK_PER_W = CTX // _NW

_CHUNK = 3136
_TAIL = VOCAB - (_NW - 1) * _CHUNK

TR = 32


def _sc_prep(W, x, table):
    mesh = plsc.VectorSubcoreMesh(core_axis_name="c", subcore_axis_name="s")

    @functools.partial(
        pl.kernel,
        mesh=mesh,
        out_type=(
            jax.ShapeDtypeStruct((CTX, D), jnp.float32),
            jax.ShapeDtypeStruct((D, VOCAB), jnp.float32),
        ),
        scratch_types=[
            pltpu.VMEM((_TOK_PER_W,), jnp.int32),
            pltpu.VMEM((_TOK_PER_W, D), jnp.float32),
            pltpu.VMEM((_CHUNK, D), jnp.float32),
            pltpu.VMEM((D, _CHUNK), jnp.float32),
            pltpu.SemaphoreType.DMA,
        ],
        compiler_params=pltpu.CompilerParams(use_tc_tiling_on_sc=False, needs_layout_passes=False),
    )
    def k(w_hbm, idx_hbm, table_hbm, gath_hbm, wt_hbm, idx_v, rows_v, wc, wtc, sem):
        wid = lax.axis_index("s") * _NC + lax.axis_index("c")
        tbase = wid * _TOK_PER_W
        pltpu.sync_copy(idx_hbm.at[pl.ds(tbase, _TOK_PER_W)], idx_v)
        gather = pltpu.async_copy(table_hbm.at[idx_v], rows_v, sem)

        cbase = wid * _CHUNK
        lanes = lax.iota(jnp.int32, 16)

        def transpose_rows(n):
            pltpu.sync_copy(w_hbm.at[pl.ds(cbase, n)], wc.at[pl.ds(0, n)])
            rsel = lax.shift_right_logical(lanes, 3)
            csel = lax.bitwise_and(lanes, 7)

            def body(r, _):
                rr = rsel + 2 * r
                v = plsc.load_gather(wc, [rr, csel])
                plsc.store_scatter(wtc, [csel, rr], v)
                return 0

            lax.fori_loop(0, n // 2, body, 0, unroll=4)
            for dd in range(D):
                pltpu.sync_copy(
                    wtc.at[dd, pl.ds(0, n)], wt_hbm.at[dd, pl.ds(cbase, n)]
                )

        @pl.when(wid < _NW - 1)
        def _():
            transpose_rows(_CHUNK)

        @pl.when(wid == _NW - 1)
        def _():
            transpose_rows(_TAIL)

        gather.wait()
        pltpu.sync_copy(rows_v, gath_hbm.at[pl.ds(tbase, _TOK_PER_W)])

    return k(W, x, table)


def _tc_unembed(gathered, pos, wt, b):
    def body(g_ref, p_ref, wt_ref, b_ref, o_ref):
        h = g_ref[...] + p_ref[...]
        o_ref[...] = (
            jnp.dot(h, wt_ref[...], preferred_element_type=jnp.float32)
            + b_ref[...][None, :]
        )

    return pl.pallas_call(
        body,
        grid=(CTX // TR,),
        in_specs=[
            pl.BlockSpec((TR, D), lambda i: (i, 0)),
            pl.BlockSpec((TR, D), lambda i: (i, 0)),
            pl.BlockSpec((D, VOCAB), lambda i: (0, 0)),
            pl.BlockSpec((VOCAB,), lambda i: (0,)),
        ],
        out_specs=pl.BlockSpec((TR, VOCAB), lambda i: (i, 0)),
        out_shape=jax.ShapeDtypeStruct((CTX, VOCAB), jnp.float32),
    )(gathered, pos, wt, b)


def kernel(x, info_embedding, position_embedding, W, b):
    gathered, wt = _sc_prep(W, x, info_embedding)
    return _tc_unembed(gathered, position_embedding, wt, b)


# speedup vs baseline: 2.2761x; 2.2761x over previous
import functools

import jax
import jax.numpy as jnp
from jax import lax
from jax.experimental import pallas as pl
from jax.experimental.pallas import tpu as pltpu
from jax.experimental.pallas import tpu_sc as plsc

VOCAB = 100000
CTX = 512
D = 8
DP = 16

_NC, _NS = 2, 16
_NW = _NC * _NS
_TOK_PER_W = CTX // _NW

VC = 4096


def _sc_embed_t(table_p, x):
    mesh = plsc.VectorSubcoreMesh(core_axis_name="c", subcore_axis_name="s")

    @functools.partial(
        pl.kernel,
        mesh=mesh,
        out_type=jax.ShapeDtypeStruct((D + 1, CTX), jnp.float32),
        scratch_types=[
            pltpu.VMEM((_TOK_PER_W,), jnp.int32),
            pltpu.VMEM((_TOK_PER_W, DP), jnp.float32),
            pltpu.VMEM((D + 1, _TOK_PER_W), jnp.float32),
            pltpu.SemaphoreType.DMA,
        ],
        compiler_params=pltpu.CompilerParams(
            use_tc_tiling_on_sc=False, needs_layout_passes=False
        ),
    )
    def k(table_hbm, idx_hbm, gt_hbm, idx_v, rows_v, gt_v, sem):
        wid = lax.axis_index("s") * _NC + lax.axis_index("c")
        tbase = wid * _TOK_PER_W
        pltpu.sync_copy(idx_hbm.at[pl.ds(tbase, _TOK_PER_W)], idx_v)
        pltpu.async_copy(table_hbm.at[idx_v], rows_v, sem).wait()
        lanes = lax.iota(jnp.int32, 16)
        for dd in range(D):
            gt_v[dd] = plsc.load_gather(rows_v, [lanes, jnp.full((16,), dd, jnp.int32)])
        gt_v[D] = jnp.full((16,), 1.0, jnp.float32)
        for dd in range(D + 1):
            pltpu.sync_copy(gt_v.at[dd], gt_hbm.at[dd, pl.ds(tbase, _TOK_PER_W)])

    return k(table_p, x)


def _tc_unembed_t(wt, b, gt, post):
    def body(wt_ref, b_ref, gt_ref, pt_ref, o_ref):
        lhs = jnp.concatenate([wt_ref[...], b_ref[...][None, :]], axis=0)
        pos9 = jnp.concatenate(
            [pt_ref[...], jnp.zeros((1, CTX), jnp.float32)], axis=0
        )
        rhs = gt_ref[...] + pos9
        o_ref[...] = lax.dot_general(
            lhs,
            rhs,
            dimension_numbers=(((0,), (0,)), ((), ())),
            preferred_element_type=jnp.float32,
        )

    return pl.pallas_call(
        body,
        grid=(pl.cdiv(VOCAB, VC),),
        in_specs=[
            pl.BlockSpec((D, VC), lambda i: (0, i)),
            pl.BlockSpec((VC,), lambda i: (i,)),
            pl.BlockSpec((D + 1, CTX), lambda i: (0, 0)),
            pl.BlockSpec((D, CTX), lambda i: (0, 0)),
        ],
        out_specs=pl.BlockSpec((VC, CTX), lambda i: (i, 0)),
        out_shape=jax.ShapeDtypeStruct((VOCAB, CTX), jnp.float32),
    )(wt, b, gt, post)


def kernel(x, info_embedding, position_embedding, W, b):
    table_p = jnp.pad(info_embedding, ((0, 0), (0, DP - D)))
    gt = _sc_embed_t(table_p, x)
    out_t = _tc_unembed_t(W.T, b, gt, position_embedding.T)
    return out_t.T


# speedup vs baseline: 4.4122x; 1.9385x over previous
import functools

import jax
import jax.numpy as jnp
from jax import lax
from jax.experimental import pallas as pl
from jax.experimental.pallas import tpu as pltpu
from jax.experimental.pallas import tpu_sc as plsc

VOCAB = 100000
CTX = 512
D = 8
DP = 16

_NC, _NS = 2, 16
_NW = _NC * _NS
_TOK_PER_W = CTX // _NW

VC = 4096
_STRIDE = 102400


def _tc_detile(table_t):
    def body(t_ref, o_ref):
        o_ref[pl.ds(0, VOCAB)] = t_ref[pl.program_id(0)]

    return pl.pallas_call(
        body,
        grid=(D,),
        in_specs=[pl.BlockSpec((D, VOCAB), lambda i: (0, 0))],
        out_specs=pl.BlockSpec((_STRIDE,), lambda i: (i,)),
        out_shape=jax.ShapeDtypeStruct((D * _STRIDE,), jnp.float32),
    )(table_t)


def _sc_embed_t(flat_t, x):
    mesh = plsc.VectorSubcoreMesh(core_axis_name="c", subcore_axis_name="s")

    @functools.partial(
        pl.kernel,
        mesh=mesh,
        out_type=jax.ShapeDtypeStruct((D + 1, CTX), jnp.float32),
        scratch_types=[
            pltpu.VMEM((_TOK_PER_W,), jnp.int32),
            pltpu.VMEM((D * _TOK_PER_W,), jnp.int32),
            pltpu.VMEM((D * _TOK_PER_W,), jnp.float32),
            pltpu.VMEM((_TOK_PER_W,), jnp.float32),
            pltpu.SemaphoreType.DMA,
        ],
        compiler_params=pltpu.CompilerParams(
            use_tc_tiling_on_sc=False, needs_layout_passes=False
        ),
    )
    def k(flat_hbm, idx_hbm, gt_hbm, idx_v, iall_v, g_v, ones_v, sem):
        wid = lax.axis_index("s") * _NC + lax.axis_index("c")
        tbase = wid * _TOK_PER_W
        pltpu.sync_copy(idx_hbm.at[pl.ds(tbase, _TOK_PER_W)], idx_v)
        xv = idx_v[...]
        for dd in range(D):
            iall_v[pl.ds(dd * _TOK_PER_W, _TOK_PER_W)] = xv + dd * _STRIDE
        pltpu.async_copy(flat_hbm.at[iall_v], g_v, sem).wait()
        ones_v[...] = jnp.full((_TOK_PER_W,), 1.0, jnp.float32)
        for dd in range(D):
            pltpu.sync_copy(
                g_v.at[pl.ds(dd * _TOK_PER_W, _TOK_PER_W)],
                gt_hbm.at[dd, pl.ds(tbase, _TOK_PER_W)],
            )
        pltpu.sync_copy(ones_v, gt_hbm.at[D, pl.ds(tbase, _TOK_PER_W)])

    return k(flat_t, x)


def _tc_unembed_t(wt, b, gt, post):
    def body(wt_ref, b_ref, gt_ref, pt_ref, o_ref):
        lhs = jnp.concatenate([wt_ref[...], b_ref[...][None, :]], axis=0)
        pos9 = jnp.concatenate(
            [pt_ref[...], jnp.zeros((1, CTX), jnp.float32)], axis=0
        )
        rhs = gt_ref[...] + pos9
        o_ref[...] = lax.dot_general(
            lhs,
            rhs,
            dimension_numbers=(((0,), (0,)), ((), ())),
            preferred_element_type=jnp.float32,
        )

    return pl.pallas_call(
        body,
        grid=(pl.cdiv(VOCAB, VC),),
        in_specs=[
            pl.BlockSpec((D, VC), lambda i: (0, i)),
            pl.BlockSpec((VC,), lambda i: (i,)),
            pl.BlockSpec((D + 1, CTX), lambda i: (0, 0)),
            pl.BlockSpec((D, CTX), lambda i: (0, 0)),
        ],
        out_specs=pl.BlockSpec((VC, CTX), lambda i: (i, 0)),
        out_shape=jax.ShapeDtypeStruct((VOCAB, CTX), jnp.float32),
    )(wt, b, gt, post)


def kernel(x, info_embedding, position_embedding, W, b):
    flat_t = _tc_detile(info_embedding.T)
    gt = _sc_embed_t(flat_t, x)
    out_t = _tc_unembed_t(W.T, b, gt, position_embedding.T)
    return out_t.T
